# Initial kernel scaffold; baseline (speedup 1.0000x reference)
import jax
import jax.numpy as jnp
from jax.experimental import pallas as pl


def kernel(kv_states, indices):
    raise NotImplementedError("write your pallas kernel here")


# speedup vs baseline: 19.6409x; 19.6409x over previous
import functools

import jax
import jax.numpy as jnp
from jax import lax
from jax.experimental import pallas as pl
from jax.experimental.pallas import tpu as pltpu
from jax.experimental.pallas import tpu_sc as plsc

NC = 2
NS = 16
NW = NC * NS
LANES = 16
CH = 128


def _build(B, H, T_kv, T_q, n_sel, D):
    rows_total = B * H * T_q * n_sel
    rows_per_w = rows_total // NW
    group_rows = T_q * n_sel
    groups_per_w = rows_per_w // group_rows
    chunks_per_w = rows_per_w // CH
    chunks_per_group = group_rows // CH

    mesh = plsc.VectorSubcoreMesh(core_axis_name="c", subcore_axis_name="s")

    @functools.partial(
        pl.kernel,
        mesh=mesh,
        out_type=jax.ShapeDtypeStruct((rows_total, D), jnp.float32),
        scratch_types=[
            pltpu.VMEM((CH,), jnp.int32),
            pltpu.VMEM((CH, D), jnp.float32),
            pltpu.SemaphoreType.DMA,
        ],
    )
    def gather_kernel(kv_hbm, idx_hbm, out_hbm, idx_v, rows_v, gsem):
        wid = lax.axis_index("s") * NC + lax.axis_index("c")
        w_row0 = wid * rows_per_w

        def chunk_body(j, carry):
            g = j // chunks_per_group
            base = (wid * groups_per_w + g) * T_kv
            row0 = w_row0 + j * CH
            pltpu.sync_copy(idx_hbm.at[pl.ds(row0, CH)], idx_v)
            bvec = jnp.broadcast_to(base.astype(jnp.int32), (LANES,))
            for k in range(CH // LANES):
                sl = pl.ds(LANES * k, LANES)
                idx_v[sl] = idx_v[sl] + bvec
            pltpu.async_copy(kv_hbm.at[idx_v], rows_v, gsem).wait()
            pltpu.sync_copy(rows_v, out_hbm.at[pl.ds(row0, CH)])
            return carry

        lax.fori_loop(0, chunks_per_w, chunk_body, 0)

    return gather_kernel


def kernel(kv_states, indices):
    B, H, T_kv, D = kv_states.shape
    _, _, T_q, n_sel = indices.shape
    kv_flat = kv_states.reshape(B * H * T_kv, D)
    idx_flat = indices.reshape(-1).astype(jnp.int32)
    out = _build(B, H, T_kv, T_q, n_sel, D)(kv_flat, idx_flat)
    return out.reshape(B, H, T_q, n_sel, D)


# speedup vs baseline: 27.0256x; 1.3760x over previous
import functools

import jax
import jax.numpy as jnp
from jax import lax
from jax.experimental import pallas as pl
from jax.experimental.pallas import tpu as pltpu
from jax.experimental.pallas import tpu_sc as plsc

NC = 2
NS = 16
NW = NC * NS
LANES = 16
CH = 128


def _build(B, H, T_kv, T_q, n_sel, D):
    rows_total = B * H * T_q * n_sel
    rows_per_w = rows_total // NW
    group_rows = T_q * n_sel
    groups_per_w = rows_per_w // group_rows
    n = rows_per_w // CH
    chunks_per_group = group_rows // CH

    mesh = plsc.VectorSubcoreMesh(core_axis_name="c", subcore_axis_name="s")

    @functools.partial(
        pl.kernel,
        mesh=mesh,
        out_type=jax.ShapeDtypeStruct((rows_total, D), jnp.float32),
        scratch_types=[
            pltpu.VMEM((CH,), jnp.int32),
            pltpu.VMEM((CH,), jnp.int32),
            pltpu.VMEM((CH, D), jnp.float32),
            pltpu.VMEM((CH, D), jnp.float32),
            pltpu.SemaphoreType.DMA,
            pltpu.SemaphoreType.DMA,
            pltpu.SemaphoreType.DMA,
            pltpu.SemaphoreType.DMA,
            pltpu.SemaphoreType.DMA,
            pltpu.SemaphoreType.DMA,
        ],
    )
    def gather_kernel(kv_hbm, idx_hbm, out_hbm,
                      idx_a, idx_b, rows_a, rows_b,
                      gsem_a, gsem_b, ssem_a, ssem_b, isem_a, isem_b):
        wid = lax.axis_index("s") * NC + lax.axis_index("c")
        w_row0 = wid * rows_per_w
        idx_bufs = (idx_a, idx_b)
        rows_bufs = (rows_a, rows_b)
        gsems = (gsem_a, gsem_b)
        ssems = (ssem_a, ssem_b)
        isems = (isem_a, isem_b)

        def row0_of(j):
            return w_row0 + j * CH

        def idx_load(j, b):
            pltpu.async_copy(idx_hbm.at[pl.ds(row0_of(j), CH)],
                             idx_bufs[b], isems[b])

        def idx_wait(b):
            pltpu.make_async_copy(idx_hbm.at[pl.ds(w_row0, CH)],
                                  idx_bufs[b], isems[b]).wait()

        def rebase(j, b):
            base = ((wid * groups_per_w + j // chunks_per_group) * T_kv)
            bvec = jnp.broadcast_to(jnp.int32(0) + base, (LANES,))
            ref = idx_bufs[b]
            for k in range(CH // LANES):
                sl = pl.ds(LANES * k, LANES)
                ref[sl] = ref[sl] + bvec

        def gather(b):
            pltpu.async_copy(kv_hbm.at[idx_bufs[b]], rows_bufs[b], gsems[b])

        def gather_wait(b):
            pltpu.make_async_copy(kv_hbm.at[idx_bufs[b]],
                                  rows_bufs[b], gsems[b]).wait()

        def store(j, b):
            pltpu.async_copy(rows_bufs[b],
                             out_hbm.at[pl.ds(row0_of(j), CH)], ssems[b])

        def store_wait(b):
            pltpu.make_async_copy(rows_bufs[b],
                                  out_hbm.at[pl.ds(w_row0, CH)],
                                  ssems[b]).wait()

        idx_load(0, 0)
        idx_load(1, 1)
        idx_wait(0)
        rebase(0, 0)
        gather(0)
        gather_wait(0)
        store(0, 0)
        idx_load(2, 0)
        idx_wait(1)
        rebase(1, 1)
        gather(1)

        def half(j, b, ob):
            gather_wait(b)
            store(j, b)
            idx_load(jnp.minimum(j + 2, n - 1), b)
            idx_wait(ob)
            rebase(j + 1, ob)
            store_wait(ob)
            gather(ob)

        def pair(j2, carry):
            half(2 * j2 + 1, 1, 0)
            half(2 * j2 + 2, 0, 1)
            return carry

        lax.fori_loop(0, (n - 2) // 2, pair, 0)

        gather_wait(1)
        store(n - 1, 1)
        idx_wait(0)
        store_wait(0)
        store_wait(1)

    return gather_kernel


def kernel(kv_states, indices):
    B, H, T_kv, D = kv_states.shape
    _, _, T_q, n_sel = indices.shape
    kv_flat = kv_states.reshape(B * H * T_kv, D)
    idx_flat = indices.reshape(-1).astype(jnp.int32)
    out = _build(B, H, T_kv, T_q, n_sel, D)(kv_flat, idx_flat)
    return out.reshape(B, H, T_q, n_sel, D)


# speedup vs baseline: 34.1074x; 1.2620x over previous
import functools

import jax
import jax.numpy as jnp
from jax import lax
from jax.experimental import pallas as pl
from jax.experimental.pallas import tpu as pltpu
from jax.experimental.pallas import tpu_sc as plsc

NC = 2
NS = 16
NW = NC * NS
LANES = 16
CH = 128
NBUF = 4


def _build(B, H, T_kv, T_q, n_sel, D):
    rows_total = B * H * T_q * n_sel
    rows_per_w = rows_total // NW
    group_rows = T_q * n_sel
    groups_per_w = rows_per_w // group_rows
    n = rows_per_w // CH
    chunks_per_group = group_rows // CH

    mesh = plsc.VectorSubcoreMesh(core_axis_name="c", subcore_axis_name="s")

    scratch = ([pltpu.VMEM((CH,), jnp.int32) for _ in range(NBUF)]
               + [pltpu.VMEM((CH, D), jnp.float32) for _ in range(NBUF)]
               + [pltpu.SemaphoreType.DMA for _ in range(3 * NBUF)])

    @functools.partial(
        pl.kernel,
        mesh=mesh,
        out_type=jax.ShapeDtypeStruct((rows_total, D), jnp.float32),
        scratch_types=scratch,
    )
    def gather_kernel(kv_hbm, idx_hbm, out_hbm, *sc):
        idx_bufs = sc[:NBUF]
        rows_bufs = sc[NBUF:2 * NBUF]
        gsems = sc[2 * NBUF:3 * NBUF]
        ssems = sc[3 * NBUF:4 * NBUF]
        isems = sc[4 * NBUF:5 * NBUF]

        wid = lax.axis_index("s") * NC + lax.axis_index("c")
        w_row0 = wid * rows_per_w

        def row0_of(j):
            return w_row0 + j * CH

        def idx_load(j, b):
            pltpu.async_copy(idx_hbm.at[pl.ds(row0_of(j), CH)],
                             idx_bufs[b], isems[b])

        def idx_wait(b):
            pltpu.make_async_copy(idx_hbm.at[pl.ds(w_row0, CH)],
                                  idx_bufs[b], isems[b]).wait()

        def rebase(j, b):
            base = ((wid * groups_per_w + j // chunks_per_group) * T_kv)
            bvec = jnp.broadcast_to(jnp.int32(0) + base, (LANES,))
            ref = idx_bufs[b]
            for k in range(CH // LANES):
                sl = pl.ds(LANES * k, LANES)
                ref[sl] = ref[sl] + bvec

        def gather(b):
            pltpu.async_copy(kv_hbm.at[idx_bufs[b]], rows_bufs[b], gsems[b])

        def gather_wait(b):
            pltpu.make_async_copy(kv_hbm.at[idx_bufs[b]],
                                  rows_bufs[b], gsems[b]).wait()

        def store(j, b):
            pltpu.async_copy(rows_bufs[b],
                             out_hbm.at[pl.ds(row0_of(j), CH)], ssems[b])

        def store_wait(b):
            pltpu.make_async_copy(rows_bufs[b],
                                  out_hbm.at[pl.ds(w_row0, CH)],
                                  ssems[b]).wait()

        for b in range(NBUF):
            idx_load(b, b)
        for j in range(NBUF):
            b = j % NBUF
            idx_wait(b)
            rebase(j, b)
            gather(b)
            if j >= 1:
                bm1 = (b - 1) % NBUF
                gather_wait(bm1)
                store(j - 1, bm1)
                idx_load(j - 1 + NBUF, bm1)

        def body(j, b):
            bm1 = (b - 1) % NBUF
            idx_wait(b)
            rebase(j, b)
            store_wait(b)
            gather(b)
            gather_wait(bm1)
            store(j - 1, bm1)
            idx_load(jnp.minimum(j + NBUF - 1, n - 1), bm1)

        def quad(q, carry):
            j0 = NBUF * q + NBUF
            for i in range(NBUF):
                body(j0 + i, i)
            return carry

        lax.fori_loop(0, (n - NBUF) // NBUF, quad, 0)

        last_b = (n - 1) % NBUF
        gather_wait(last_b)
        store(n - 1, last_b)
        for i in range(NBUF - 1):
            idx_wait((n + i) % NBUF)
        for b in range(NBUF):
            store_wait(b)

    return gather_kernel


def kernel(kv_states, indices):
    B, H, T_kv, D = kv_states.shape
    _, _, T_q, n_sel = indices.shape
    kv_flat = kv_states.reshape(B * H * T_kv, D)
    idx_flat = indices.reshape(-1).astype(jnp.int32)
    out = _build(B, H, T_kv, T_q, n_sel, D)(kv_flat, idx_flat)
    return out.reshape(B, H, T_q, n_sel, D)
